# Initial kernel scaffold; baseline (speedup 1.0000x reference)
import jax
import jax.numpy as jnp
from jax.experimental import pallas as pl


def kernel(x, edge_index, W1, b1, W2, b2):
    raise NotImplementedError("write your pallas kernel here")


# speedup vs baseline: 2.8532x; 2.8532x over previous
import jax
import jax.numpy as jnp
from jax.experimental import pallas as pl
from jax.experimental.pallas import tpu as pltpu

N = 10000
D = 128


def _mm_kernel(a_ref, w_ref, o_ref):
    o_ref[...] = jnp.dot(a_ref[...], w_ref[...],
                         preferred_element_type=jnp.float32)


def _mm(a, w):
    return pl.pallas_call(
        _mm_kernel,
        out_shape=jax.ShapeDtypeStruct((a.shape[0], w.shape[1]), jnp.float32),
    )(a, w)


def kernel(x, edge_index, W1, b1, W2, b2):
    src = edge_index[0].astype(jnp.int32)
    dst = edge_index[1].astype(jnp.int32)
    ones = jnp.ones((src.shape[0],), jnp.float32)
    deg = jnp.ones((N,), jnp.float32).at[dst].add(ones)
    dis = jax.lax.rsqrt(deg)

    def layer(h, W, b):
        s = _mm(h, W) * dis[:, None]
        agg = s.at[dst].add(s[src])
        return agg * dis[:, None] + b

    h = jax.nn.relu(layer(x, W1, b1))
    return layer(h, W2, b2)


# speedup vs baseline: 10.0081x; 3.5076x over previous
import jax
import jax.numpy as jnp
from jax import lax
from jax.experimental import pallas as pl
from jax.experimental.pallas import tpu as pltpu
from jax.experimental.pallas import tpu_sc as plsc

N = 10000
D = 128
NC = 2
NS = 16
NW = NC * NS
CHUNK = 128
NPH = 2
KH = 40
KD = 80
NBUF = 2
EPAD = NW * NPH * KH * CHUNK
NP = 10112
RPT = NP // NS
_RCHUNKS = ((0, 128), (128, 128), (256, 128), (384, 128), (512, 120))


def _deg_body(dst_hbm, out_hbm, acc, dst_v, ones_v, bounce, dsem):
    core = lax.axis_index("c")
    sub = lax.axis_index("s")
    wid = core * NS + sub

    def fill_ones(i, c):
        ones_v[i, :] = jnp.ones((16,), jnp.float32)
        return c
    lax.fori_loop(0, CHUNK, fill_ones, 0)

    def fill_zero(i, c):
        bounce[i, :] = jnp.zeros((16,), jnp.float32)
        return c
    lax.fori_loop(0, RPT, fill_zero, 0)

    dbase = sub * RPT
    pltpu.sync_copy(bounce, acc.at[pl.ds(dbase, RPT)])
    pltpu.sync_copy(dst_hbm.at[wid], dst_v)
    plsc.subcore_barrier()

    def round_body(r, c):
        for b in range(16):
            j = r * 16 + b
            pltpu.async_copy(ones_v, acc.at[dst_v.at[j]], dsem, add=True)
        for b in range(16):
            pltpu.make_async_copy(ones_v, acc.at[dst_v.at[0]], dsem).wait()
        return c
    lax.fori_loop(0, KD // 16, round_body, 0)

    plsc.subcore_barrier()
    pltpu.sync_copy(acc.at[pl.ds(dbase, RPT)], bounce)
    pltpu.sync_copy(bounce, out_hbm.at[core, pl.ds(dbase, RPT)])


_deg = pl.kernel(
    _deg_body,
    out_type=jax.ShapeDtypeStruct((NC, NP, 16), jnp.float32),
    mesh=plsc.VectorSubcoreMesh(core_axis_name="c", subcore_axis_name="s"),
    scratch_types=[
        pltpu.VMEM_SHARED((NP, 16), jnp.float32),
        pltpu.VMEM((KD, CHUNK), jnp.int32),
        pltpu.VMEM((CHUNK, 16), jnp.float32),
        pltpu.VMEM((RPT, 16), jnp.float32),
        pltpu.SemaphoreType.DMA,
    ],
)


def _edge_body(s_hbm, src_hbm, dst_hbm, out_hbm,
               acc, src_v, dst_v, buf, g0, g1, s0, s1):
    core = lax.axis_index("c")
    sub = lax.axis_index("s")
    gsems = (g0, g1)
    ssems = (s0, s1)

    base = sub * RPT
    for off, sz in _RCHUNKS:
        rs = pl.ds(base + off, sz)
        bs = buf.at[0, pl.ds(0, sz)]
        pltpu.sync_copy(s_hbm.at[rs], bs)
        pltpu.sync_copy(bs, acc.at[rs])
    plsc.subcore_barrier()

    def gather(j, b):
        pltpu.async_copy(s_hbm.at[src_v.at[j]], buf.at[b], gsems[b])

    for p in range(NPH):
        pltpu.sync_copy(src_hbm.at[core, sub, p], src_v)
        pltpu.sync_copy(dst_hbm.at[core, sub, p], dst_v)
        for b in range(NBUF):
            gather(b, b)

        def round_body(o, c):
            for b in range(NBUF):
                j = o * NBUF + b
                pltpu.make_async_copy(s_hbm.at[src_v.at[j]], buf.at[b],
                                      gsems[b]).wait()
                pltpu.async_copy(buf.at[b], acc.at[dst_v.at[j]], ssems[b],
                                 add=True)
            for b in range(NBUF):
                j = o * NBUF + b
                pltpu.make_async_copy(buf.at[b], acc.at[dst_v.at[j]],
                                      ssems[b]).wait()

                @pl.when(o < KH // NBUF - 1)
                def _():
                    gather(j + NBUF, b)
            return c

        lax.fori_loop(0, KH // NBUF, round_body, 0)

    plsc.subcore_barrier()
    for off, sz in _RCHUNKS:
        rs = pl.ds(base + off, sz)
        bs = buf.at[0, pl.ds(0, sz)]
        pltpu.sync_copy(acc.at[rs], bs)
        pltpu.sync_copy(bs, out_hbm.at[core, rs])


_edge = pl.kernel(
    _edge_body,
    out_type=jax.ShapeDtypeStruct((NC, NP, D), jnp.float32),
    mesh=plsc.VectorSubcoreMesh(core_axis_name="c", subcore_axis_name="s"),
    scratch_types=[
        pltpu.VMEM_SHARED((NP, D), jnp.float32),
        pltpu.VMEM((KH, CHUNK), jnp.int32),
        pltpu.VMEM((KH, CHUNK), jnp.int32),
        pltpu.VMEM((NBUF, CHUNK, D), jnp.float32),
    ] + [pltpu.SemaphoreType.DMA] * 4,
)


def _dis_from_counts(cnt_ref):
    c = cnt_ref[0, :, 0:1] + cnt_ref[1, :, 0:1] + 1.0
    return lax.rsqrt(c)


def _tc1_body(cnt_ref, x_ref, w_ref, o_ref):
    dis = _dis_from_counts(cnt_ref)
    o_ref[...] = jnp.dot(x_ref[...], w_ref[...],
                         preferred_element_type=jnp.float32) * dis


def _tc2_body(acc_ref, s_ref, cnt_ref, b_ref, w_ref, o_ref):
    dis = _dis_from_counts(cnt_ref)
    g = acc_ref[0] + acc_ref[1] - s_ref[...]
    h = jnp.maximum(g * dis + b_ref[...], 0.0)
    o_ref[...] = jnp.dot(h, w_ref[...],
                         preferred_element_type=jnp.float32) * dis


def _tc3_body(acc_ref, s_ref, cnt_ref, b_ref, o_ref):
    c = cnt_ref[0, 0:N, 0:1] + cnt_ref[1, 0:N, 0:1] + 1.0
    dis = lax.rsqrt(c)
    g = acc_ref[0, 0:N, :] + acc_ref[1, 0:N, :] - s_ref[0:N, :]
    o_ref[...] = g * dis + b_ref[...]


_tc1 = pl.pallas_call(
    _tc1_body, out_shape=jax.ShapeDtypeStruct((NP, D), jnp.float32))
_tc2 = pl.pallas_call(
    _tc2_body, out_shape=jax.ShapeDtypeStruct((NP, D), jnp.float32))
_tc3 = pl.pallas_call(
    _tc3_body, out_shape=jax.ShapeDtypeStruct((N, D), jnp.float32))


def kernel(x, edge_index, W1, b1, W2, b2):
    src = edge_index[0].astype(jnp.int32)
    dst = edge_index[1].astype(jnp.int32)
    pad = jnp.full((EPAD - src.shape[0],), N, jnp.int32)
    srcp = jnp.concatenate([src, pad]).reshape(NC, NS, NPH, KH, CHUNK)
    dstf = jnp.concatenate([dst, pad])
    dstp = dstf.reshape(NC, NS, NPH, KH, CHUNK)
    dstw = dstf.reshape(NW, KD, CHUNK)
    xp = jnp.concatenate([x, jnp.zeros((NP - N, D), jnp.float32)], axis=0)
    b1r = b1.reshape(1, D)
    b2r = b2.reshape(1, D)

    cnt = _deg(dstw)
    s1 = _tc1(cnt, xp, W1)
    a1 = _edge(s1, srcp, dstp)
    s2 = _tc2(a1, s1, cnt, b1r, W2)
    a2 = _edge(s2, srcp, dstp)
    return _tc3(a2, s2, cnt, b2r)


# speedup vs baseline: 25.4101x; 2.5389x over previous
import jax
import jax.numpy as jnp
from jax import lax
from jax.experimental import pallas as pl
from jax.experimental.pallas import tpu as pltpu
from jax.experimental.pallas import tpu_sc as plsc

N = 10000
D = 128
NC = 2
NS = 16
NW = NC * NS
CHUNK = 128
NPH = 2
KH = 40
KD = 80
NBUF = 2
EPAD = NW * NPH * KH * CHUNK
NP = 10112
RPT = NP // NS
_RCHUNKS = ((0, 128), (128, 128), (256, 128), (384, 128), (512, 120))


def _deg_body(dst_hbm, out_hbm, acc, dst_v, ones_v, bounce, dsem):
    core = lax.axis_index("c")
    sub = lax.axis_index("s")
    wid = core * NS + sub

    def fill_ones(i, c):
        ones_v[i, :] = jnp.ones((16,), jnp.float32)
        return c
    lax.fori_loop(0, CHUNK, fill_ones, 0)

    def fill_zero(i, c):
        bounce[i, :] = jnp.zeros((16,), jnp.float32)
        return c
    lax.fori_loop(0, RPT, fill_zero, 0)

    dbase = sub * RPT
    pltpu.sync_copy(bounce, acc.at[pl.ds(dbase, RPT)])
    pltpu.sync_copy(dst_hbm.at[wid], dst_v)
    plsc.subcore_barrier()

    def round_body(r, c):
        for b in range(16):
            j = r * 16 + b
            pltpu.async_copy(ones_v, acc.at[dst_v.at[j]], dsem, add=True)
        for b in range(16):
            pltpu.make_async_copy(ones_v, acc.at[dst_v.at[0]], dsem).wait()
        return c
    lax.fori_loop(0, KD // 16, round_body, 0)

    plsc.subcore_barrier()
    pltpu.sync_copy(acc.at[pl.ds(dbase, RPT)], bounce)
    pltpu.sync_copy(bounce, out_hbm.at[core, pl.ds(dbase, RPT)])


_deg = pl.kernel(
    _deg_body,
    out_type=jax.ShapeDtypeStruct((NC, NP, 16), jnp.float32),
    mesh=plsc.VectorSubcoreMesh(core_axis_name="c", subcore_axis_name="s"),
    scratch_types=[
        pltpu.VMEM_SHARED((NP, 16), jnp.float32),
        pltpu.VMEM((KD, CHUNK), jnp.int32),
        pltpu.VMEM((CHUNK, 16), jnp.float32),
        pltpu.VMEM((RPT, 16), jnp.float32),
        pltpu.SemaphoreType.DMA,
    ],
)


def _edge_body(s_hbm, src_hbm, dst_hbm, out_hbm,
               acc, src_v, dst_v, buf, g0, g1, s0, s1):
    core = lax.axis_index("c")
    sub = lax.axis_index("s")
    gsems = (g0, g1)
    ssems = (s0, s1)

    base = sub * RPT
    for off, sz in _RCHUNKS:
        rs = pl.ds(base + off, sz)
        bs = buf.at[0, pl.ds(0, sz)]
        pltpu.sync_copy(s_hbm.at[rs], bs)
        pltpu.sync_copy(bs, acc.at[rs])
    plsc.subcore_barrier()

    def gather(j, b):
        pltpu.async_copy(s_hbm.at[src_v.at[j]], buf.at[b], gsems[b])

    for p in range(NPH):
        pltpu.sync_copy(src_hbm.at[core, sub, p], src_v)
        pltpu.sync_copy(dst_hbm.at[core, sub, p], dst_v)
        for b in range(NBUF):
            gather(b, b)

        def round_body(o, c):
            for b in range(NBUF):
                j = o * NBUF + b
                pltpu.make_async_copy(s_hbm.at[src_v.at[j]], buf.at[b],
                                      gsems[b]).wait()
                pltpu.async_copy(buf.at[b], acc.at[dst_v.at[j]], ssems[b],
                                 add=True)
            for b in range(NBUF):
                j = o * NBUF + b
                pltpu.make_async_copy(buf.at[b], acc.at[dst_v.at[j]],
                                      ssems[b]).wait()

                @pl.when(o < KH // NBUF - 1)
                def _():
                    gather(j + NBUF, b)
            return c

        lax.fori_loop(0, KH // NBUF, round_body, 0)

    plsc.subcore_barrier()
    for off, sz in _RCHUNKS:
        rs = pl.ds(base + off, sz)
        bs = buf.at[0, pl.ds(0, sz)]
        pltpu.sync_copy(acc.at[rs], bs)
        pltpu.sync_copy(bs, out_hbm.at[core, rs])


_edge = pl.kernel(
    _edge_body,
    out_type=jax.ShapeDtypeStruct((NC, NP, D), jnp.float32),
    mesh=plsc.VectorSubcoreMesh(core_axis_name="c", subcore_axis_name="s"),
    scratch_types=[
        pltpu.VMEM_SHARED((NP, D), jnp.float32),
        pltpu.VMEM((KH, CHUNK), jnp.int32),
        pltpu.VMEM((KH, CHUNK), jnp.int32),
        pltpu.VMEM((NBUF, CHUNK, D), jnp.float32),
    ] + [pltpu.SemaphoreType.DMA] * 4,
)


def _dis_from_counts(cnt_ref):
    c = cnt_ref[0, :, 0:1] + cnt_ref[1, :, 0:1] + 1.0
    return lax.rsqrt(c)


def _tc1_body(cnt_ref, x_ref, w_ref, o_ref):
    dis = _dis_from_counts(cnt_ref)
    o_ref[...] = jnp.dot(x_ref[...], w_ref[...],
                         preferred_element_type=jnp.float32) * dis


def _tc2_body(acc_ref, s_ref, cnt_ref, b_ref, w_ref, o_ref):
    dis = _dis_from_counts(cnt_ref)
    g = acc_ref[0] + acc_ref[1] - s_ref[...]
    h = jnp.maximum(g * dis + b_ref[...], 0.0)
    o_ref[...] = jnp.dot(h, w_ref[...],
                         preferred_element_type=jnp.float32) * dis


def _tc3_body(acc_ref, s_ref, cnt_ref, b_ref, o_ref):
    c = cnt_ref[0, 0:N, 0:1] + cnt_ref[1, 0:N, 0:1] + 1.0
    dis = lax.rsqrt(c)
    g = acc_ref[0, 0:N, :] + acc_ref[1, 0:N, :] - s_ref[0:N, :]
    o_ref[...] = g * dis + b_ref[...]


_tc1 = pl.pallas_call(
    _tc1_body, out_shape=jax.ShapeDtypeStruct((NP, D), jnp.float32))
_tc2 = pl.pallas_call(
    _tc2_body, out_shape=jax.ShapeDtypeStruct((NP, D), jnp.float32))
_tc3 = pl.pallas_call(
    _tc3_body, out_shape=jax.ShapeDtypeStruct((N, D), jnp.float32))


def kernel(x, edge_index, W1, b1, W2, b2):
    src = edge_index[0].astype(jnp.int32)
    dst = edge_index[1].astype(jnp.int32)
    pad = N + jnp.arange(EPAD - src.shape[0], dtype=jnp.int32) % (NP - N)
    srcp = jnp.concatenate([src, pad]).reshape(NC, NS, NPH, KH, CHUNK)
    dstf = jnp.concatenate([dst, pad])
    dstp = dstf.reshape(NC, NS, NPH, KH, CHUNK)
    dstw = dstf.reshape(NW, KD, CHUNK)
    xp = jnp.concatenate([x, jnp.zeros((NP - N, D), jnp.float32)], axis=0)
    b1r = b1.reshape(1, D)
    b2r = b2.reshape(1, D)

    cnt = _deg(dstw)
    s1 = _tc1(cnt, xp, W1)
    a1 = _edge(s1, srcp, dstp)
    s2 = _tc2(a1, s1, cnt, b1r, W2)
    a2 = _edge(s2, srcp, dstp)
    return _tc3(a2, s2, cnt, b2r)


# speedup vs baseline: 32.0294x; 1.2605x over previous
import jax
import jax.numpy as jnp
from jax import lax
from jax.experimental import pallas as pl
from jax.experimental.pallas import tpu as pltpu
from jax.experimental.pallas import tpu_sc as plsc

N = 10000
D = 128
NC = 2
NS = 16
NW = NC * NS
CHUNK = 128
KE = 84
KD = 80
NBUF = 3
NIB = 6
EPAD = NW * KE * CHUNK
EPD = NW * KD * CHUNK
NP = 10112
NR = 10040
RPT = NP // NS
_C632 = ((0, 128), (128, 128), (256, 128), (384, 128), (512, 120))
_C608 = ((0, 128), (128, 128), (256, 128), (384, 128), (512, 96))


def _deg_body(dst_hbm, out_hbm, acc, dst_v, ones_v, bounce, dsem):
    core = lax.axis_index("c")
    sub = lax.axis_index("s")
    wid = core * NS + sub

    def fill_ones(i, c):
        ones_v[i, :] = jnp.ones((16,), jnp.float32)
        return c
    lax.fori_loop(0, CHUNK, fill_ones, 0)

    def fill_zero(i, c):
        bounce[i, :] = jnp.zeros((16,), jnp.float32)
        return c
    lax.fori_loop(0, RPT, fill_zero, 0)

    dbase = sub * RPT
    pltpu.sync_copy(bounce, acc.at[pl.ds(dbase, RPT)])
    pltpu.sync_copy(dst_hbm.at[wid], dst_v)
    plsc.subcore_barrier()

    def round_body(r, c):
        for b in range(16):
            j = r * 16 + b
            pltpu.async_copy(ones_v, acc.at[dst_v.at[j]], dsem, add=True)
        for b in range(16):
            pltpu.make_async_copy(ones_v, acc.at[dst_v.at[0]], dsem).wait()
        return c
    lax.fori_loop(0, KD // 16, round_body, 0)

    plsc.subcore_barrier()
    pltpu.sync_copy(acc.at[pl.ds(dbase, RPT)], bounce)
    pltpu.sync_copy(bounce, out_hbm.at[core, pl.ds(dbase, RPT)])


_deg = pl.kernel(
    _deg_body,
    out_type=jax.ShapeDtypeStruct((NC, NP, 16), jnp.float32),
    mesh=plsc.VectorSubcoreMesh(core_axis_name="c", subcore_axis_name="s"),
    scratch_types=[
        pltpu.VMEM_SHARED((NP, 16), jnp.float32),
        pltpu.VMEM((KD, CHUNK), jnp.int32),
        pltpu.VMEM((CHUNK, 16), jnp.float32),
        pltpu.VMEM((RPT, 16), jnp.float32),
        pltpu.SemaphoreType.DMA,
    ],
)


def _edge_body(s_hbm, eidx_hbm, out_hbm,
               acc, ib, buf, g0, g1, g2, s0, s1, s2, isem):
    core = lax.axis_index("c")
    sub = lax.axis_index("s")
    gsems = (g0, g1, g2)
    ssems = (s0, s1, s2)

    def initio(chunks, base, write):
        for off, sz in chunks:
            rs = pl.ds(base + off, sz)
            bs = buf.at[0, pl.ds(0, sz)]
            if write:
                pltpu.sync_copy(acc.at[rs], bs)
                pltpu.sync_copy(bs, out_hbm.at[core, rs])
            else:
                pltpu.sync_copy(s_hbm.at[rs], bs)
                pltpu.sync_copy(bs, acc.at[rs])

    def tile_io(write):
        @pl.when(sub < 13)
        def _():
            initio(_C632, sub * 632, write)

        @pl.when(sub >= 13)
        def _():
            initio(_C608, 8216 + (sub - 13) * 608, write)

    tile_io(False)

    for q in range(NIB):
        pltpu.async_copy(eidx_hbm.at[core, sub, q], ib.at[q], isem)

    def wait_idx():
        pltpu.make_async_copy(eidx_hbm.at[core, sub, 0], ib.at[0],
                              isem).wait()

    def gather_start(slot, b):
        pltpu.async_copy(s_hbm.at[ib.at[slot, 0]], buf.at[b], gsems[b])

    def gather_wait(slot, b):
        pltpu.make_async_copy(s_hbm.at[ib.at[slot, 0]], buf.at[b],
                              gsems[b]).wait()

    def scat_start(slot, b):
        pltpu.async_copy(buf.at[b], acc.at[ib.at[slot, 1]], ssems[b],
                         add=True)

    def scat_wait(slot, b):
        pltpu.make_async_copy(buf.at[b], acc.at[ib.at[slot, 1]],
                              ssems[b]).wait()

    plsc.subcore_barrier()

    wait_idx()
    wait_idx()
    gather_start(0, 0)
    gather_start(1, 1)

    NRND = KE // NIB

    def round_body(o, c):
        for b in range(NIB):
            bb = b % NBUF
            pb = (b - 1) % NBUF
            gather_wait(b, bb)
            scat_start(b, bb)
            if b == 0:
                @pl.when(o > 0)
                def _():
                    scat_wait((b - 1) % NIB, pb)
            else:
                scat_wait(b - 1, pb)
            if b < 4:
                wait_idx()
                gather_start((b + 2) % NIB, pb)
            else:
                @pl.when(o < NRND - 1)
                def _():
                    wait_idx()
                    gather_start((b + 2) % NIB, pb)
            if b == 0:
                @pl.when(o > 0)
                def _():
                    pltpu.async_copy(
                        eidx_hbm.at[core, sub, o * NIB + b + 5],
                        ib.at[(b - 1) % NIB], isem)
            else:
                @pl.when(o < NRND - 1)
                def _():
                    pltpu.async_copy(
                        eidx_hbm.at[core, sub, o * NIB + b + 5],
                        ib.at[b - 1], isem)
        return c

    lax.fori_loop(0, NRND, round_body, 0)
    scat_wait(NIB - 1, (KE - 1) % NBUF)
    plsc.subcore_barrier()

    tile_io(True)


_edge = pl.kernel(
    _edge_body,
    out_type=jax.ShapeDtypeStruct((NC, NR, D), jnp.float32),
    mesh=plsc.VectorSubcoreMesh(core_axis_name="c", subcore_axis_name="s"),
    scratch_types=[
        pltpu.VMEM_SHARED((NR, D), jnp.float32),
        pltpu.VMEM((NIB, 2, CHUNK), jnp.int32),
        pltpu.VMEM((NBUF, CHUNK, D), jnp.float32),
    ] + [pltpu.SemaphoreType.DMA] * 7,
)


def _dis_from_counts(cnt_ref):
    c = cnt_ref[0, :, 0:1] + cnt_ref[1, :, 0:1] + 1.0
    return lax.rsqrt(c)


def _mm1_body(x_ref, w_ref, o_ref):
    o_ref[...] = jnp.dot(x_ref[...], w_ref[...],
                         preferred_element_type=jnp.float32)


def _sc1_body(u_ref, cnt_ref, o_ref):
    o_ref[...] = u_ref[...] * _dis_from_counts(cnt_ref)


def _tc2_body(acc_ref, s_ref, cnt_ref, b_ref, w_ref, o_ref):
    dis = _dis_from_counts(cnt_ref)
    g = acc_ref[0, 0:N, :] + acc_ref[1, 0:N, :] - s_ref[0:N, :]
    gp = jnp.concatenate(
        [g, jnp.zeros((NP - N, D), jnp.float32)], axis=0)
    h = jnp.maximum(gp * dis + b_ref[...], 0.0)
    o_ref[...] = jnp.dot(h, w_ref[...],
                         preferred_element_type=jnp.float32) * dis


def _tc3_body(acc_ref, s_ref, cnt_ref, b_ref, o_ref):
    c = cnt_ref[0, 0:N, 0:1] + cnt_ref[1, 0:N, 0:1] + 1.0
    dis = lax.rsqrt(c)
    g = acc_ref[0, 0:N, :] + acc_ref[1, 0:N, :] - s_ref[0:N, :]
    o_ref[...] = g * dis + b_ref[...]


_mm1 = pl.pallas_call(
    _mm1_body, out_shape=jax.ShapeDtypeStruct((NP, D), jnp.float32))
_sc1 = pl.pallas_call(
    _sc1_body, out_shape=jax.ShapeDtypeStruct((NP, D), jnp.float32))
_tc2 = pl.pallas_call(
    _tc2_body, out_shape=jax.ShapeDtypeStruct((NP, D), jnp.float32))
_tc3 = pl.pallas_call(
    _tc3_body, out_shape=jax.ShapeDtypeStruct((N, D), jnp.float32))


def kernel(x, edge_index, W1, b1, W2, b2):
    src = edge_index[0].astype(jnp.int32)
    dst = edge_index[1].astype(jnp.int32)
    pade = N + jnp.arange(EPAD - src.shape[0], dtype=jnp.int32) % (NR - N)
    srcf = jnp.concatenate([src, pade]).reshape(NC, NS, KE, CHUNK)
    dstf = jnp.concatenate([dst, pade]).reshape(NC, NS, KE, CHUNK)
    eidx = jnp.stack([srcf, dstf], axis=3)
    padd = N + jnp.arange(EPD - src.shape[0], dtype=jnp.int32) % (NR - N)
    dstw = jnp.concatenate([dst, padd]).reshape(NW, KD, CHUNK)
    xp = jnp.concatenate([x, jnp.zeros((NP - N, D), jnp.float32)], axis=0)
    b1r = b1.reshape(1, D)
    b2r = b2.reshape(1, D)

    cnt = _deg(dstw)
    u1 = _mm1(xp, W1)
    s1 = _sc1(u1, cnt)
    a1 = _edge(s1, eidx)
    s2 = _tc2(a1, s1, cnt, b1r, W2)
    a2 = _edge(s2, eidx)
    return _tc3(a2, s2, cnt, b2r)
